# Initial kernel scaffold; baseline (speedup 1.0000x reference)
import jax
import jax.numpy as jnp
from jax.experimental import pallas as pl


def kernel(z_loga_expert):
    raise NotImplementedError("write your pallas kernel here")


# speedup vs baseline: 38.9753x; 38.9753x over previous
import jax
import jax.numpy as jnp
from jax.experimental import pallas as pl
from jax.experimental.pallas import tpu as pltpu

_NL, _NE, _N = 32, 8, 14336
_K = 7168
_ROWS = _NL * _NE
_BR = 8

_HI0 = 0x7F800000


def _body(x_ref, o_ref):
    x = x_ref[...]
    v = jnp.maximum(x, 0.0)
    u = jax.lax.bitcast_convert_type(v, jnp.int32)

    lo = jnp.zeros((_BR, 1), jnp.int32)
    hi = jnp.full((_BR, 1), _HI0, jnp.int32)

    def it(_, carry):
        lo, hi = carry
        mid = lo + (hi - lo) // 2
        cnt = jnp.sum((u <= mid).astype(jnp.int32), axis=1, keepdims=True)
        pred = cnt >= _K
        return jnp.where(pred, lo, mid + 1), jnp.where(pred, mid, hi)

    lo, hi = jax.lax.fori_loop(0, 31, it, (lo, hi))
    o_ref[...] = jnp.where(u <= lo, 0.0, v)


def kernel(z_loga_expert):
    flat = z_loga_expert.reshape(_ROWS, _N)
    out = pl.pallas_call(
        _body,
        grid=(_ROWS // _BR,),
        in_specs=[pl.BlockSpec((_BR, _N), lambda i: (i, 0))],
        out_specs=pl.BlockSpec((_BR, _N), lambda i: (i, 0)),
        out_shape=jax.ShapeDtypeStruct((_ROWS, _N), jnp.float32),
        compiler_params=pltpu.CompilerParams(
            dimension_semantics=("arbitrary",),
        ),
    )(flat)
    return out.reshape(_NL, _NE, _N)


# speedup vs baseline: 109.7667x; 2.8163x over previous
import jax
import jax.numpy as jnp
from jax.experimental import pallas as pl
from jax.experimental.pallas import tpu as pltpu

_NL, _NE, _N = 32, 8, 14336
_K = 7168
_ROWS = _NL * _NE
_BR = 32


def _body(x_ref, o_ref, u_ref):
    x = x_ref[...]
    v = jnp.maximum(x, 0.0)
    u = jax.lax.bitcast_convert_type(v, jnp.int32)
    u_ref[...] = u

    lo = jnp.min(u, axis=1, keepdims=True)
    hi = jnp.max(u, axis=1, keepdims=True)

    def cond(carry):
        lo, hi = carry
        return jnp.any(lo < hi)

    def it(carry):
        lo, hi = carry
        mid = lo + (hi - lo) // 2
        cnt = jnp.sum((u_ref[...] <= mid).astype(jnp.int32), axis=1,
                      keepdims=True)
        pred = cnt >= _K
        return jnp.where(pred, lo, mid + 1), jnp.where(pred, mid, hi)

    lo, hi = jax.lax.while_loop(cond, it, (lo, hi))
    o_ref[...] = jnp.where(u_ref[...] <= lo, 0.0, v)


def kernel(z_loga_expert):
    flat = z_loga_expert.reshape(_ROWS, _N)
    out = pl.pallas_call(
        _body,
        grid=(_ROWS // _BR,),
        in_specs=[pl.BlockSpec((_BR, _N), lambda i: (i, 0))],
        out_specs=pl.BlockSpec((_BR, _N), lambda i: (i, 0)),
        out_shape=jax.ShapeDtypeStruct((_ROWS, _N), jnp.float32),
        scratch_shapes=[pltpu.VMEM((_BR, _N), jnp.int32)],
        compiler_params=pltpu.CompilerParams(
            dimension_semantics=("arbitrary",),
        ),
    )(flat)
    return out.reshape(_NL, _NE, _N)
